# baseline (speedup 1.0000x reference)
import jax
import jax.numpy as jnp
from jax.experimental import pallas as pl
from jax.experimental.pallas import tpu as pltpu


_BM = 512
_LAG = 1


def _mlp_block(x_ref, w1a_ref, w1b_hbm, b1_ref, w2_hbm, b2_ref, o_ref,
               h_scr, w1b_scr, w2_scr, w1b_sem, w2_sem):
    i = pl.program_id(0)
    nsteps = pl.num_programs(0)
    half = w1a_ref.shape[0]
    n1 = w1b_hbm.shape[0]
    w1b_copy = pltpu.make_async_copy(
        w1b_hbm.at[pl.ds(half, n1 - half), :], w1b_scr, w1b_sem)
    w2_copy = pltpu.make_async_copy(w2_hbm, w2_scr, w2_sem)

    @pl.when(i == 0)
    def _start_dmas():
        w1b_copy.start()
        w2_copy.start()

    @pl.when(i < nsteps - _LAG)
    def _layer1():
        xb = x_ref[...].astype(jnp.bfloat16)
        ha = jax.lax.dot_general(
            xb, w1a_ref[...], (((1,), (1,)), ((), ())),
            preferred_element_type=jnp.float32)
        h_scr[i % (_LAG + 1), :, :half] = jnp.maximum(
            ha + b1_ref[:, :half], 0.0).astype(jnp.bfloat16)

        @pl.when(i == 0)
        def _wait_w1b():
            w1b_copy.wait()

        hb = jax.lax.dot_general(
            xb, w1b_scr[...], (((1,), (1,)), ((), ())),
            preferred_element_type=jnp.float32)
        h_scr[i % (_LAG + 1), :, half:] = jnp.maximum(
            hb + b1_ref[:, half:], 0.0).astype(jnp.bfloat16)

    @pl.when(i == _LAG)
    def _wait_w2():
        w2_copy.wait()

    @pl.when(i >= _LAG)
    def _layer2():
        hin = h_scr[(i - _LAG) % (_LAG + 1)]
        o = jax.lax.dot_general(
            hin, w2_scr[...], (((1,), (1,)), ((), ())),
            preferred_element_type=jnp.float32)
        o_ref[...] = o + b2_ref[...]


def kernel(input, W1, b1, W2, b2):
    M, K = input.shape
    N1, _ = W1.shape
    N2, _ = W2.shape
    half = N1 // 2
    nblocks = M // _BM
    grid = (nblocks + _LAG,)
    last = nblocks - 1
    return pl.pallas_call(
        _mlp_block,
        grid=grid,
        in_specs=[
            pl.BlockSpec((_BM, K), lambda i: (jnp.minimum(i, last), 0)),
            pl.BlockSpec((half, K), lambda i: (0, 0)),
            pl.BlockSpec(memory_space=pl.ANY),
            pl.BlockSpec((1, N1), lambda i: (0, 0)),
            pl.BlockSpec(memory_space=pl.ANY),
            pl.BlockSpec((1, N2), lambda i: (0, 0)),
        ],
        out_specs=pl.BlockSpec((_BM, N2), lambda i: (jnp.maximum(i - _LAG, 0), 0)),
        out_shape=jax.ShapeDtypeStruct((M, N2), jnp.float32),
        scratch_shapes=[
            pltpu.VMEM((_LAG + 1, _BM, N1), jnp.bfloat16),
            pltpu.VMEM((N1 - half, K), jnp.float32),
            pltpu.VMEM((N2, N1), jnp.float32),
            pltpu.SemaphoreType.DMA,
            pltpu.SemaphoreType.DMA,
        ],
        compiler_params=pltpu.CompilerParams(
            vmem_limit_bytes=63 * 1024 * 1024),
    )(input, W1, W1, b1.reshape(1, N1), W2, b2.reshape(1, N2))
